# baseline (speedup 1.0000x reference)
import functools

import jax
import jax.numpy as jnp
from jax.experimental import pallas as pl
from jax.experimental.pallas import tpu as pltpu
from jax.experimental.pallas import tpu_sc as plsc

_N = 10000
_NP = 10240
_E = 160000
_K = 128
_NW = 32
_NCHUNK = 5120 // _K
_EP = _NW * _NCHUNK * _K
_RPS = _NP // 16

_mesh = plsc.VectorSubcoreMesh(core_axis_name="c", subcore_axis_name="s")
_sc_params = pltpu.CompilerParams(use_tc_tiling_on_sc=False)


def _make_deg():
    @functools.partial(
        pl.kernel,
        out_type=jax.ShapeDtypeStruct((2, _NP, 16), jnp.float32),
        mesh=_mesh,
        scratch_types=[
            pltpu.VMEM((_NCHUNK, _K), jnp.int32),
            pltpu.VMEM((_K, 16), jnp.float32),
            pltpu.VMEM_SHARED((_NP, 16), jnp.float32),
        ],
        compiler_params=_sc_params,
    )
    def deg(dst_hbm, ones_hbm, zeros_hbm, out_hbm, dstv, onesv, acc):
        c = jax.lax.axis_index("c")
        s = jax.lax.axis_index("s")
        wid = s * 2 + c
        base = s * _RPS
        pltpu.sync_copy(zeros_hbm.at[pl.ds(base, _RPS)],
                        acc.at[pl.ds(base, _RPS)])
        pltpu.sync_copy(dst_hbm.at[wid], dstv)
        pltpu.sync_copy(ones_hbm, onesv)
        plsc.subcore_barrier()

        @pl.loop(0, _NCHUNK)
        def _(j):
            pltpu.sync_copy(onesv, acc.at[dstv.at[j]], add=True)

        plsc.subcore_barrier()
        pltpu.sync_copy(acc.at[pl.ds(base, _RPS)],
                        out_hbm.at[c, pl.ds(base, _RPS)])

    return deg


def _edge_stream(table, acc, srcv, dstv, rows0, rows1, sem0, sem1):
    @pl.loop(0, _NCHUNK, step=2)
    def _(j):
        pltpu.sync_copy(table.at[srcv.at[j]], rows0)

        @pl.when(j > 0)
        def _():
            pltpu.make_async_copy(rows1, acc.at[dstv.at[j - 1]], sem1).wait()

        pltpu.async_copy(rows0, acc.at[dstv.at[j]], sem0, add=True)
        pltpu.sync_copy(table.at[srcv.at[j + 1]], rows1)
        pltpu.make_async_copy(rows0, acc.at[dstv.at[j]], sem0).wait()
        pltpu.async_copy(rows1, acc.at[dstv.at[j + 1]], sem1, add=True)

    pltpu.make_async_copy(rows1, acc.at[dstv.at[_NCHUNK - 1]], sem1).wait()


def _make_propagate(width, spmem_src=False):
    scratch = [
        pltpu.VMEM((_NCHUNK, _K), jnp.int32),
        pltpu.VMEM((_NCHUNK, _K), jnp.int32),
        pltpu.VMEM((_K, width), jnp.float32),
        pltpu.VMEM((_K, width), jnp.float32),
        pltpu.VMEM_SHARED((_NP, width), jnp.float32),
        pltpu.SemaphoreType.DMA,
        pltpu.SemaphoreType.DMA,
    ]
    if spmem_src:
        scratch.append(pltpu.VMEM_SHARED((_NP, width), jnp.float32))

    @functools.partial(
        pl.kernel,
        out_type=jax.ShapeDtypeStruct((2, _NP, width), jnp.float32),
        mesh=_mesh,
        scratch_types=scratch,
        compiler_params=_sc_params,
    )
    def prop(y_hbm, src_hbm, dst_hbm, out_hbm, srcv, dstv, rows0, rows1,
             acc, sem0, sem1, *maybe_ysp):
        c = jax.lax.axis_index("c")
        s = jax.lax.axis_index("s")
        wid = s * 2 + c
        base = s * _RPS
        pltpu.sync_copy(y_hbm.at[pl.ds(base, _RPS)],
                        acc.at[pl.ds(base, _RPS)])
        if spmem_src:
            ysp = maybe_ysp[0]
            pltpu.sync_copy(y_hbm.at[pl.ds(base, _RPS)],
                            ysp.at[pl.ds(base, _RPS)])
            src_tab = ysp
        else:
            src_tab = y_hbm
        pltpu.sync_copy(src_hbm.at[wid], srcv)
        pltpu.sync_copy(dst_hbm.at[wid], dstv)
        plsc.subcore_barrier()
        _edge_stream(src_tab, acc, srcv, dstv, rows0, rows1, sem0, sem1)
        plsc.subcore_barrier()
        pltpu.sync_copy(acc.at[pl.ds(base, _RPS)],
                        out_hbm.at[c, pl.ds(base, _RPS)])

    return prop


def _make_prop128_split():
    half = jax.ShapeDtypeStruct((2, _NP, 64), jnp.float32)

    @functools.partial(
        pl.kernel,
        out_type=(half, half),
        mesh=_mesh,
        scratch_types=[
            pltpu.VMEM((_NCHUNK, _K), jnp.int32),
            pltpu.VMEM((_NCHUNK, _K), jnp.int32),
            pltpu.VMEM((_K, 64), jnp.float32),
            pltpu.VMEM((_K, 64), jnp.float32),
            pltpu.VMEM_SHARED((_NP, 64), jnp.float32),
            pltpu.VMEM_SHARED((_NP, 64), jnp.float32),
            pltpu.SemaphoreType.DMA,
            pltpu.SemaphoreType.DMA,
        ],
        compiler_params=_sc_params,
    )
    def prop(y0_hbm, y1_hbm, src_hbm, dst_hbm, o0_hbm, o1_hbm,
             srcv, dstv, rows0, rows1, acc, ysp, sem0, sem1):
        c = jax.lax.axis_index("c")
        s = jax.lax.axis_index("s")
        wid = s * 2 + c
        stripe = pl.ds(s * _RPS, _RPS)
        pltpu.sync_copy(src_hbm.at[wid], srcv)
        pltpu.sync_copy(dst_hbm.at[wid], dstv)
        for p, (yh, oh) in enumerate(((y0_hbm, o0_hbm), (y1_hbm, o1_hbm))):
            pltpu.sync_copy(yh.at[stripe], acc.at[stripe])
            pltpu.sync_copy(yh.at[stripe], ysp.at[stripe])
            plsc.subcore_barrier()
            _edge_stream(ysp, acc, srcv, dstv, rows0, rows1, sem0, sem1)
            plsc.subcore_barrier()
            pltpu.sync_copy(acc.at[stripe], oh.at[c, stripe])
            if p == 0:
                plsc.subcore_barrier()

    return prop


_deg_kernel = _make_deg()
_prop128 = _make_prop128_split()
_prop32 = _make_propagate(32, spmem_src=True)


_BROW = 512


def _mmscale_body(x_ref, w_ref, d0_ref, d1_ref, y0_ref, y1_ref, dinv_ref):
    dinv = jax.lax.rsqrt(d0_ref[...] + d1_ref[...] + 1.0)
    dinv_ref[...] = dinv
    xw = jnp.dot(x_ref[...], w_ref[...], preferred_element_type=jnp.float32)
    y = xw * dinv[:, 0:1]
    y0_ref[...] = y[:, :64]
    y1_ref[...] = y[:, 64:]


def _mmscale(x, w, d0, d1):
    return pl.pallas_call(
        _mmscale_body,
        grid=(_NP // _BROW,),
        in_specs=[pl.BlockSpec((_BROW, 128), lambda i: (i, 0)),
                  pl.BlockSpec((128, 128), lambda i: (0, 0)),
                  pl.BlockSpec((_BROW, 16), lambda i: (i, 0)),
                  pl.BlockSpec((_BROW, 16), lambda i: (i, 0))],
        out_specs=[pl.BlockSpec((_BROW, 64), lambda i: (i, 0)),
                   pl.BlockSpec((_BROW, 64), lambda i: (i, 0)),
                   pl.BlockSpec((_BROW, 16), lambda i: (i, 0))],
        out_shape=[jax.ShapeDtypeStruct((_NP, 64), jnp.float32),
                   jax.ShapeDtypeStruct((_NP, 64), jnp.float32),
                   jax.ShapeDtypeStruct((_NP, 16), jnp.float32)],
        compiler_params=pltpu.CompilerParams(
            dimension_semantics=("parallel",)),
    )(x, w, d0, d1)


def _combine1_body(a00_ref, a01_ref, a10_ref, a11_ref, y0_ref, y1_ref,
                   dinv_ref, b_ref, w_ref, y2_ref):
    dinv = dinv_ref[:, 0:1]
    h0 = dinv * (a00_ref[...] + a01_ref[...] - y0_ref[...]) + b_ref[:, :64]
    h1 = dinv * (a10_ref[...] + a11_ref[...] - y1_ref[...]) + b_ref[:, 64:]
    h = jnp.maximum(jnp.concatenate([h0, h1], axis=1), 0.0)
    y2_ref[...] = jnp.dot(h, w_ref[...],
                          preferred_element_type=jnp.float32) * dinv


def _combine1(p0, p1, y10, y11, dinv, b1, w_cat):
    half = pl.BlockSpec((_BROW, 64), lambda i: (i, 0))
    return pl.pallas_call(
        _combine1_body,
        grid=(_NP // _BROW,),
        in_specs=[half, half, half, half, half, half,
                  pl.BlockSpec((_BROW, 16), lambda i: (i, 0)),
                  pl.BlockSpec((1, 128), lambda i: (0, 0)),
                  pl.BlockSpec((128, 32), lambda i: (0, 0))],
        out_specs=pl.BlockSpec((_BROW, 32), lambda i: (i, 0)),
        out_shape=jax.ShapeDtypeStruct((_NP, 32), jnp.float32),
        compiler_params=pltpu.CompilerParams(
            dimension_semantics=("parallel",)),
    )(p0[0], p0[1], p1[0], p1[1], y10, y11, dinv, b1, w_cat)


def _combine2_body(a0_ref, a1_ref, y_ref, dinv_ref, b_ref, mu_ref, lv_ref):
    dinv = dinv_ref[:, 0:1]
    o = dinv * (a0_ref[...] + a1_ref[...] - y_ref[...]) + b_ref[...]
    mu_ref[...] = o[:, :16]
    lv_ref[...] = o[:, 16:]


def _combine2(a0, a1, y2, dinv, b_cat):
    half = pl.BlockSpec((_BROW, 16), lambda i: (i, 0))
    return pl.pallas_call(
        _combine2_body,
        grid=(_NP // _BROW,),
        in_specs=[pl.BlockSpec((_BROW, 32), lambda i: (i, 0)),
                  pl.BlockSpec((_BROW, 32), lambda i: (i, 0)),
                  pl.BlockSpec((_BROW, 32), lambda i: (i, 0)),
                  pl.BlockSpec((_BROW, 16), lambda i: (i, 0)),
                  pl.BlockSpec((1, 32), lambda i: (0, 0))],
        out_specs=[half, half],
        out_shape=[jax.ShapeDtypeStruct((_NP, 16), jnp.float32),
                   jax.ShapeDtypeStruct((_NP, 16), jnp.float32)],
        compiler_params=pltpu.CompilerParams(
            dimension_semantics=("parallel",)),
    )(a0, a1, y2, dinv, b_cat)


_BM = 512
_BN = 2048


def _decoder_body(z_row_ref, z_col_ref, out_ref):
    a = z_row_ref[...] * -1.0
    nx = jax.lax.dot_general(a, z_col_ref[...],
                             (((1,), (1,)), ((), ())),
                             preferred_element_type=jnp.float32)
    nx = jnp.clip(nx, -30.0, 30.0)
    d = 1.0 + jnp.exp(nx)
    r = jax.lax.bitcast_convert_type(
        jnp.int32(0x7EF127EA) - jax.lax.bitcast_convert_type(d, jnp.int32),
        jnp.float32)
    r = r * (2.0 - d * r)
    r = r * (2.0 - d * r)
    out_ref[...] = r


def _decoder(z):
    return pl.pallas_call(
        _decoder_body,
        grid=(pl.cdiv(_N, _BM), pl.cdiv(_N, _BN)),
        in_specs=[pl.BlockSpec((_BM, 16), lambda i, j: (i, 0)),
                  pl.BlockSpec((_BN, 16), lambda i, j: (j, 0))],
        out_specs=pl.BlockSpec((_BM, _BN), lambda i, j: (i, j)),
        out_shape=jax.ShapeDtypeStruct((_N, _N), jnp.float32),
        compiler_params=pltpu.CompilerParams(
            dimension_semantics=("parallel", "parallel")),
    )(z, z)


def kernel(x, edge_index, W1, b1, W_mu, b_mu, W_lv, b_lv):
    ei = edge_index.astype(jnp.int32)
    src = jnp.concatenate([ei[0], jnp.zeros((_EP - _E,), jnp.int32)])
    dst = jnp.concatenate([ei[1],
                           jnp.full((_EP - _E,), _N, jnp.int32)])
    src3 = src.reshape(_NW, _NCHUNK, _K)
    dst3 = dst.reshape(_NW, _NCHUNK, _K)

    ones = jnp.ones((_K, 16), jnp.float32)
    zeros = jnp.zeros((_NP, 16), jnp.float32)
    x_pad = jnp.pad(x, ((0, _NP - _N), (0, 0)))
    w_cat = jnp.concatenate([W_mu, W_lv], axis=1)
    b_cat = jnp.concatenate([b_mu, b_lv]).reshape(1, 32)

    deg = _deg_kernel(dst3, ones, zeros)
    y10, y11, dinv = _mmscale(x_pad, W1, deg[0], deg[1])
    p0, p1 = _prop128(y10, y11, src3, dst3)
    y2 = _combine1(p0, p1, y10, y11, dinv, b1.reshape(1, 128), w_cat)
    p2 = _prop32(y2, src3, dst3)
    mu_p, lv_p = _combine2(p2[0], p2[1], y2, dinv, b_cat)
    adj_pred = _decoder(mu_p)
    return (adj_pred, mu_p[:_N], lv_p[:_N])


# speedup vs baseline: 1.0736x; 1.0736x over previous
import functools

import jax
import jax.numpy as jnp
from jax.experimental import pallas as pl
from jax.experimental.pallas import tpu as pltpu
from jax.experimental.pallas import tpu_sc as plsc

_N = 10000
_NP = 10240
_E = 160000
_K = 128
_NW = 32
_NCHUNK = 5120 // _K
_EP = _NW * _NCHUNK * _K
_RPS = _NP // 16

_mesh = plsc.VectorSubcoreMesh(core_axis_name="c", subcore_axis_name="s")
_sc_params = pltpu.CompilerParams(use_tc_tiling_on_sc=False)


def _make_deg():
    @functools.partial(
        pl.kernel,
        out_type=jax.ShapeDtypeStruct((2, _NP, 16), jnp.float32),
        mesh=_mesh,
        scratch_types=[
            pltpu.VMEM((_NCHUNK, _K), jnp.int32),
            pltpu.VMEM((_K, 16), jnp.float32),
            pltpu.VMEM_SHARED((_NP, 16), jnp.float32),
        ],
        compiler_params=_sc_params,
    )
    def deg(dst_hbm, ones_hbm, zeros_hbm, out_hbm, dstv, onesv, acc):
        c = jax.lax.axis_index("c")
        s = jax.lax.axis_index("s")
        wid = s * 2 + c
        base = s * _RPS
        pltpu.sync_copy(zeros_hbm.at[pl.ds(base, _RPS)],
                        acc.at[pl.ds(base, _RPS)])
        pltpu.sync_copy(dst_hbm.at[wid], dstv)
        pltpu.sync_copy(ones_hbm, onesv)
        plsc.subcore_barrier()

        @pl.loop(0, _NCHUNK)
        def _(j):
            pltpu.sync_copy(onesv, acc.at[dstv.at[j]], add=True)

        plsc.subcore_barrier()
        pltpu.sync_copy(acc.at[pl.ds(base, _RPS)],
                        out_hbm.at[c, pl.ds(base, _RPS)])

    return deg


def _edge_stream(table, acc, srcv, dstv, rows0, rows1, sem0, sem1):
    @pl.loop(0, _NCHUNK, step=2)
    def _(j):
        pltpu.sync_copy(table.at[srcv.at[j]], rows0)

        @pl.when(j > 0)
        def _():
            pltpu.make_async_copy(rows1, acc.at[dstv.at[j - 1]], sem1).wait()

        pltpu.async_copy(rows0, acc.at[dstv.at[j]], sem0, add=True)
        pltpu.sync_copy(table.at[srcv.at[j + 1]], rows1)
        pltpu.make_async_copy(rows0, acc.at[dstv.at[j]], sem0).wait()
        pltpu.async_copy(rows1, acc.at[dstv.at[j + 1]], sem1, add=True)

    pltpu.make_async_copy(rows1, acc.at[dstv.at[_NCHUNK - 1]], sem1).wait()


def _make_propagate(width, spmem_src=False):
    scratch = [
        pltpu.VMEM((_NCHUNK, _K), jnp.int32),
        pltpu.VMEM((_NCHUNK, _K), jnp.int32),
        pltpu.VMEM((_K, width), jnp.float32),
        pltpu.VMEM((_K, width), jnp.float32),
        pltpu.VMEM_SHARED((_NP, width), jnp.float32),
        pltpu.SemaphoreType.DMA,
        pltpu.SemaphoreType.DMA,
    ]
    if spmem_src:
        scratch.append(pltpu.VMEM_SHARED((_NP, width), jnp.float32))

    @functools.partial(
        pl.kernel,
        out_type=jax.ShapeDtypeStruct((2, _NP, width), jnp.float32),
        mesh=_mesh,
        scratch_types=scratch,
        compiler_params=_sc_params,
    )
    def prop(y_hbm, src_hbm, dst_hbm, out_hbm, srcv, dstv, rows0, rows1,
             acc, sem0, sem1, *maybe_ysp):
        c = jax.lax.axis_index("c")
        s = jax.lax.axis_index("s")
        wid = s * 2 + c
        base = s * _RPS
        pltpu.sync_copy(y_hbm.at[pl.ds(base, _RPS)],
                        acc.at[pl.ds(base, _RPS)])
        if spmem_src:
            ysp = maybe_ysp[0]
            pltpu.sync_copy(y_hbm.at[pl.ds(base, _RPS)],
                            ysp.at[pl.ds(base, _RPS)])
            src_tab = ysp
        else:
            src_tab = y_hbm
        pltpu.sync_copy(src_hbm.at[wid], srcv)
        pltpu.sync_copy(dst_hbm.at[wid], dstv)
        plsc.subcore_barrier()
        _edge_stream(src_tab, acc, srcv, dstv, rows0, rows1, sem0, sem1)
        plsc.subcore_barrier()
        pltpu.sync_copy(acc.at[pl.ds(base, _RPS)],
                        out_hbm.at[c, pl.ds(base, _RPS)])

    return prop


def _make_prop128_split():
    half = jax.ShapeDtypeStruct((2, _NP, 64), jnp.float32)

    @functools.partial(
        pl.kernel,
        out_type=(half, half),
        mesh=_mesh,
        scratch_types=[
            pltpu.VMEM((_NCHUNK, _K), jnp.int32),
            pltpu.VMEM((_NCHUNK, _K), jnp.int32),
            pltpu.VMEM((_K, 64), jnp.float32),
            pltpu.VMEM((_K, 64), jnp.float32),
            pltpu.VMEM_SHARED((_NP, 64), jnp.float32),
            pltpu.VMEM_SHARED((_NP, 64), jnp.float32),
            pltpu.SemaphoreType.DMA,
            pltpu.SemaphoreType.DMA,
        ],
        compiler_params=_sc_params,
    )
    def prop(y0_hbm, y1_hbm, src_hbm, dst_hbm, o0_hbm, o1_hbm,
             srcv, dstv, rows0, rows1, acc, ysp, sem0, sem1):
        c = jax.lax.axis_index("c")
        s = jax.lax.axis_index("s")
        wid = s * 2 + c
        stripe = pl.ds(s * _RPS, _RPS)
        pltpu.sync_copy(src_hbm.at[wid], srcv)
        pltpu.sync_copy(dst_hbm.at[wid], dstv)
        for p, (yh, oh) in enumerate(((y0_hbm, o0_hbm), (y1_hbm, o1_hbm))):
            pltpu.sync_copy(yh.at[stripe], acc.at[stripe])
            pltpu.sync_copy(yh.at[stripe], ysp.at[stripe])
            plsc.subcore_barrier()
            _edge_stream(ysp, acc, srcv, dstv, rows0, rows1, sem0, sem1)
            plsc.subcore_barrier()
            pltpu.sync_copy(acc.at[stripe], oh.at[c, stripe])
            if p == 0:
                plsc.subcore_barrier()

    return prop


_deg_kernel = _make_deg()
_prop128 = _make_prop128_split()
_prop32 = _make_propagate(32, spmem_src=True)


_BROW = 512


def _matmul_body(x_ref, w_ref, o_ref):
    o_ref[...] = jnp.dot(x_ref[...], w_ref[...],
                         preferred_element_type=jnp.float32)


def _matmul(x, w):
    return pl.pallas_call(
        _matmul_body,
        grid=(_NP // _BROW,),
        in_specs=[pl.BlockSpec((_BROW, 128), lambda i: (i, 0)),
                  pl.BlockSpec((128, 128), lambda i: (0, 0))],
        out_specs=pl.BlockSpec((_BROW, 128), lambda i: (i, 0)),
        out_shape=jax.ShapeDtypeStruct((_NP, 128), jnp.float32),
        compiler_params=pltpu.CompilerParams(
            dimension_semantics=("parallel",)),
    )(x, w)


def _scale_body(xw_ref, d0_ref, d1_ref, y0_ref, y1_ref, dinv_ref):
    dinv = jax.lax.rsqrt(d0_ref[...] + d1_ref[...] + 1.0)
    dinv_ref[...] = dinv
    y = xw_ref[...] * dinv[:, 0:1]
    y0_ref[...] = y[:, :64]
    y1_ref[...] = y[:, 64:]


def _scale(xw, d0, d1):
    return pl.pallas_call(
        _scale_body,
        grid=(_NP // _BROW,),
        in_specs=[pl.BlockSpec((_BROW, 128), lambda i: (i, 0)),
                  pl.BlockSpec((_BROW, 16), lambda i: (i, 0)),
                  pl.BlockSpec((_BROW, 16), lambda i: (i, 0))],
        out_specs=[pl.BlockSpec((_BROW, 64), lambda i: (i, 0)),
                   pl.BlockSpec((_BROW, 64), lambda i: (i, 0)),
                   pl.BlockSpec((_BROW, 16), lambda i: (i, 0))],
        out_shape=[jax.ShapeDtypeStruct((_NP, 64), jnp.float32),
                   jax.ShapeDtypeStruct((_NP, 64), jnp.float32),
                   jax.ShapeDtypeStruct((_NP, 16), jnp.float32)],
        compiler_params=pltpu.CompilerParams(
            dimension_semantics=("parallel",)),
    )(xw, d0, d1)


def _combine1_body(a00_ref, a01_ref, a10_ref, a11_ref, y0_ref, y1_ref,
                   dinv_ref, b_ref, w_ref, y2_ref):
    dinv = dinv_ref[:, 0:1]
    h0 = dinv * (a00_ref[...] + a01_ref[...] - y0_ref[...]) + b_ref[:, :64]
    h1 = dinv * (a10_ref[...] + a11_ref[...] - y1_ref[...]) + b_ref[:, 64:]
    h = jnp.maximum(jnp.concatenate([h0, h1], axis=1), 0.0)
    y2_ref[...] = jnp.dot(h, w_ref[...],
                          preferred_element_type=jnp.float32) * dinv


def _combine1(p0, p1, y10, y11, dinv, b1, w_cat):
    half = pl.BlockSpec((_BROW, 64), lambda i: (i, 0))
    return pl.pallas_call(
        _combine1_body,
        grid=(_NP // _BROW,),
        in_specs=[half, half, half, half, half, half,
                  pl.BlockSpec((_BROW, 16), lambda i: (i, 0)),
                  pl.BlockSpec((1, 128), lambda i: (0, 0)),
                  pl.BlockSpec((128, 32), lambda i: (0, 0))],
        out_specs=pl.BlockSpec((_BROW, 32), lambda i: (i, 0)),
        out_shape=jax.ShapeDtypeStruct((_NP, 32), jnp.float32),
        compiler_params=pltpu.CompilerParams(
            dimension_semantics=("parallel",)),
    )(p0[0], p0[1], p1[0], p1[1], y10, y11, dinv, b1, w_cat)


def _combine2_body(a0_ref, a1_ref, y_ref, dinv_ref, b_ref, mu_ref, lv_ref):
    dinv = dinv_ref[:, 0:1]
    o = dinv * (a0_ref[...] + a1_ref[...] - y_ref[...]) + b_ref[...]
    mu_ref[...] = o[:, :16]
    lv_ref[...] = o[:, 16:]


def _combine2(a0, a1, y2, dinv, b_cat):
    half = pl.BlockSpec((_BROW, 16), lambda i: (i, 0))
    return pl.pallas_call(
        _combine2_body,
        grid=(_NP // _BROW,),
        in_specs=[pl.BlockSpec((_BROW, 32), lambda i: (i, 0)),
                  pl.BlockSpec((_BROW, 32), lambda i: (i, 0)),
                  pl.BlockSpec((_BROW, 32), lambda i: (i, 0)),
                  pl.BlockSpec((_BROW, 16), lambda i: (i, 0)),
                  pl.BlockSpec((1, 32), lambda i: (0, 0))],
        out_specs=[half, half],
        out_shape=[jax.ShapeDtypeStruct((_NP, 16), jnp.float32),
                   jax.ShapeDtypeStruct((_NP, 16), jnp.float32)],
        compiler_params=pltpu.CompilerParams(
            dimension_semantics=("parallel",)),
    )(a0, a1, y2, dinv, b_cat)


_BM = 512
_BN = 2048


def _decoder_body(z_row_ref, z_col_ref, out_ref):
    a = z_row_ref[...] * 0.5
    acc = jax.lax.dot_general(a, z_col_ref[...],
                              (((1,), (1,)), ((), ())),
                              preferred_element_type=jnp.float32)
    out_ref[...] = jnp.tanh(acc) * 0.5 + 0.5


def _decoder(z):
    return pl.pallas_call(
        _decoder_body,
        grid=(pl.cdiv(_N, _BM), pl.cdiv(_N, _BN)),
        in_specs=[pl.BlockSpec((_BM, 16), lambda i, j: (i, 0)),
                  pl.BlockSpec((_BN, 16), lambda i, j: (j, 0))],
        out_specs=pl.BlockSpec((_BM, _BN), lambda i, j: (i, j)),
        out_shape=jax.ShapeDtypeStruct((_N, _N), jnp.float32),
        compiler_params=pltpu.CompilerParams(
            dimension_semantics=("parallel", "parallel")),
    )(z, z)


def kernel(x, edge_index, W1, b1, W_mu, b_mu, W_lv, b_lv):
    ei = edge_index.astype(jnp.int32)
    src = jnp.concatenate([ei[0], jnp.zeros((_EP - _E,), jnp.int32)])
    dst = jnp.concatenate([ei[1],
                           jnp.full((_EP - _E,), _N, jnp.int32)])
    src3 = src.reshape(_NW, _NCHUNK, _K)
    dst3 = dst.reshape(_NW, _NCHUNK, _K)

    ones = jnp.ones((_K, 16), jnp.float32)
    zeros = jnp.zeros((_NP, 16), jnp.float32)
    x_pad = jnp.pad(x, ((0, _NP - _N), (0, 0)))
    w_cat = jnp.concatenate([W_mu, W_lv], axis=1)
    b_cat = jnp.concatenate([b_mu, b_lv]).reshape(1, 32)

    deg = _deg_kernel(dst3, ones, zeros)
    xw = _matmul(x_pad, W1)
    y10, y11, dinv = _scale(xw, deg[0], deg[1])
    p0, p1 = _prop128(y10, y11, src3, dst3)
    y2 = _combine1(p0, p1, y10, y11, dinv, b1.reshape(1, 128), w_cat)
    p2 = _prop32(y2, src3, dst3)
    mu_p, lv_p = _combine2(p2[0], p2[1], y2, dinv, b_cat)
    adj_pred = _decoder(mu_p)
    return (adj_pred, mu_p[:_N], lv_p[:_N])


# speedup vs baseline: 1.0829x; 1.0087x over previous
import functools

import jax
import jax.numpy as jnp
from jax.experimental import pallas as pl
from jax.experimental.pallas import tpu as pltpu
from jax.experimental.pallas import tpu_sc as plsc

_N = 10000
_NP = 10240
_E = 160000
_K = 128
_NW = 32
_NCHUNK = 5120 // _K
_EP = _NW * _NCHUNK * _K
_RPS = _NP // 16

_mesh = plsc.VectorSubcoreMesh(core_axis_name="c", subcore_axis_name="s")
_sc_params = pltpu.CompilerParams(use_tc_tiling_on_sc=False)


def _make_deg():
    @functools.partial(
        pl.kernel,
        out_type=jax.ShapeDtypeStruct((2, _NP, 16), jnp.float32),
        mesh=_mesh,
        scratch_types=[
            pltpu.VMEM((_NCHUNK, _K), jnp.int32),
            pltpu.VMEM((_K, 16), jnp.float32),
            pltpu.VMEM_SHARED((_NP, 16), jnp.float32),
        ],
        compiler_params=_sc_params,
    )
    def deg(dst_hbm, ones_hbm, zeros_hbm, out_hbm, dstv, onesv, acc):
        c = jax.lax.axis_index("c")
        s = jax.lax.axis_index("s")
        wid = s * 2 + c
        base = s * _RPS
        pltpu.sync_copy(zeros_hbm.at[pl.ds(base, _RPS)],
                        acc.at[pl.ds(base, _RPS)])
        pltpu.sync_copy(dst_hbm.at[wid], dstv)
        pltpu.sync_copy(ones_hbm, onesv)
        plsc.subcore_barrier()

        @pl.loop(0, _NCHUNK)
        def _(j):
            pltpu.sync_copy(onesv, acc.at[dstv.at[j]], add=True)

        plsc.subcore_barrier()
        pltpu.sync_copy(acc.at[pl.ds(base, _RPS)],
                        out_hbm.at[c, pl.ds(base, _RPS)])

    return deg


def _edge_stream(table, acc, srcv, dstv, rows0, rows1, sem0, sem1):
    @pl.loop(0, _NCHUNK, step=2)
    def _(j):
        pltpu.sync_copy(table.at[srcv.at[j]], rows0)

        @pl.when(j > 0)
        def _():
            pltpu.make_async_copy(rows1, acc.at[dstv.at[j - 1]], sem1).wait()

        pltpu.async_copy(rows0, acc.at[dstv.at[j]], sem0, add=True)
        pltpu.sync_copy(table.at[srcv.at[j + 1]], rows1)
        pltpu.make_async_copy(rows0, acc.at[dstv.at[j]], sem0).wait()
        pltpu.async_copy(rows1, acc.at[dstv.at[j + 1]], sem1, add=True)

    pltpu.make_async_copy(rows1, acc.at[dstv.at[_NCHUNK - 1]], sem1).wait()


def _make_propagate(width, spmem_src=False):
    scratch = [
        pltpu.VMEM((_NCHUNK, _K), jnp.int32),
        pltpu.VMEM((_NCHUNK, _K), jnp.int32),
        pltpu.VMEM((_K, width), jnp.float32),
        pltpu.VMEM((_K, width), jnp.float32),
        pltpu.VMEM_SHARED((_NP, width), jnp.float32),
        pltpu.SemaphoreType.DMA,
        pltpu.SemaphoreType.DMA,
    ]
    if spmem_src:
        scratch.append(pltpu.VMEM_SHARED((_NP, width), jnp.float32))

    @functools.partial(
        pl.kernel,
        out_type=jax.ShapeDtypeStruct((2, _NP, width), jnp.float32),
        mesh=_mesh,
        scratch_types=scratch,
        compiler_params=_sc_params,
    )
    def prop(y_hbm, src_hbm, dst_hbm, out_hbm, srcv, dstv, rows0, rows1,
             acc, sem0, sem1, *maybe_ysp):
        c = jax.lax.axis_index("c")
        s = jax.lax.axis_index("s")
        wid = s * 2 + c
        base = s * _RPS
        pltpu.sync_copy(y_hbm.at[pl.ds(base, _RPS)],
                        acc.at[pl.ds(base, _RPS)])
        if spmem_src:
            ysp = maybe_ysp[0]
            pltpu.sync_copy(y_hbm.at[pl.ds(base, _RPS)],
                            ysp.at[pl.ds(base, _RPS)])
            src_tab = ysp
        else:
            src_tab = y_hbm
        pltpu.sync_copy(src_hbm.at[wid], srcv)
        pltpu.sync_copy(dst_hbm.at[wid], dstv)
        plsc.subcore_barrier()
        _edge_stream(src_tab, acc, srcv, dstv, rows0, rows1, sem0, sem1)
        plsc.subcore_barrier()
        pltpu.sync_copy(acc.at[pl.ds(base, _RPS)],
                        out_hbm.at[c, pl.ds(base, _RPS)])

    return prop


def _make_prop128_split():
    half = jax.ShapeDtypeStruct((2, _NP, 64), jnp.float32)

    @functools.partial(
        pl.kernel,
        out_type=(half, half),
        mesh=_mesh,
        scratch_types=[
            pltpu.VMEM((_NCHUNK, _K), jnp.int32),
            pltpu.VMEM((_NCHUNK, _K), jnp.int32),
            pltpu.VMEM((_K, 64), jnp.float32),
            pltpu.VMEM((_K, 64), jnp.float32),
            pltpu.VMEM_SHARED((_NP, 64), jnp.float32),
            pltpu.VMEM_SHARED((_NP, 64), jnp.float32),
            pltpu.SemaphoreType.DMA,
            pltpu.SemaphoreType.DMA,
        ],
        compiler_params=_sc_params,
    )
    def prop(y0_hbm, y1_hbm, src_hbm, dst_hbm, o0_hbm, o1_hbm,
             srcv, dstv, rows0, rows1, acc, ysp, sem0, sem1):
        c = jax.lax.axis_index("c")
        s = jax.lax.axis_index("s")
        wid = s * 2 + c
        stripe = pl.ds(s * _RPS, _RPS)
        pltpu.sync_copy(src_hbm.at[wid], srcv)
        pltpu.sync_copy(dst_hbm.at[wid], dstv)
        for p, (yh, oh) in enumerate(((y0_hbm, o0_hbm), (y1_hbm, o1_hbm))):
            pltpu.sync_copy(yh.at[stripe], acc.at[stripe])
            pltpu.sync_copy(yh.at[stripe], ysp.at[stripe])
            plsc.subcore_barrier()
            _edge_stream(ysp, acc, srcv, dstv, rows0, rows1, sem0, sem1)
            plsc.subcore_barrier()
            pltpu.sync_copy(acc.at[stripe], oh.at[c, stripe])
            if p == 0:
                plsc.subcore_barrier()

    return prop


_deg_kernel = _make_deg()
_prop128 = _make_prop128_split()
_prop32 = _make_propagate(32, spmem_src=True)


_BROW = 512


def _mmscale_body(x_ref, w_ref, d0_ref, d1_ref, y0_ref, y1_ref, dinv_ref):
    dinv = jax.lax.rsqrt(d0_ref[...] + d1_ref[...] + 1.0)
    dinv_ref[...] = dinv
    xw = jnp.dot(x_ref[...], w_ref[...], preferred_element_type=jnp.float32)
    y = xw * dinv[:, 0:1]
    y0_ref[...] = y[:, :64]
    y1_ref[...] = y[:, 64:]


def _mmscale(x, w, d0, d1):
    return pl.pallas_call(
        _mmscale_body,
        grid=(_NP // _BROW,),
        in_specs=[pl.BlockSpec((_BROW, 128), lambda i: (i, 0)),
                  pl.BlockSpec((128, 128), lambda i: (0, 0)),
                  pl.BlockSpec((_BROW, 16), lambda i: (i, 0)),
                  pl.BlockSpec((_BROW, 16), lambda i: (i, 0))],
        out_specs=[pl.BlockSpec((_BROW, 64), lambda i: (i, 0)),
                   pl.BlockSpec((_BROW, 64), lambda i: (i, 0)),
                   pl.BlockSpec((_BROW, 16), lambda i: (i, 0))],
        out_shape=[jax.ShapeDtypeStruct((_NP, 64), jnp.float32),
                   jax.ShapeDtypeStruct((_NP, 64), jnp.float32),
                   jax.ShapeDtypeStruct((_NP, 16), jnp.float32)],
        compiler_params=pltpu.CompilerParams(
            dimension_semantics=("parallel",)),
    )(x, w, d0, d1)


def _combine1_body(a00_ref, a01_ref, a10_ref, a11_ref, y0_ref, y1_ref,
                   dinv_ref, b_ref, w_ref, y2_ref):
    dinv = dinv_ref[:, 0:1]
    h0 = dinv * (a00_ref[...] + a01_ref[...] - y0_ref[...]) + b_ref[:, :64]
    h1 = dinv * (a10_ref[...] + a11_ref[...] - y1_ref[...]) + b_ref[:, 64:]
    h = jnp.maximum(jnp.concatenate([h0, h1], axis=1), 0.0)
    y2_ref[...] = jnp.dot(h, w_ref[...],
                          preferred_element_type=jnp.float32) * dinv


def _combine1(p0, p1, y10, y11, dinv, b1, w_cat):
    half = pl.BlockSpec((_BROW, 64), lambda i: (i, 0))
    return pl.pallas_call(
        _combine1_body,
        grid=(_NP // _BROW,),
        in_specs=[half, half, half, half, half, half,
                  pl.BlockSpec((_BROW, 16), lambda i: (i, 0)),
                  pl.BlockSpec((1, 128), lambda i: (0, 0)),
                  pl.BlockSpec((128, 32), lambda i: (0, 0))],
        out_specs=pl.BlockSpec((_BROW, 32), lambda i: (i, 0)),
        out_shape=jax.ShapeDtypeStruct((_NP, 32), jnp.float32),
        compiler_params=pltpu.CompilerParams(
            dimension_semantics=("parallel",)),
    )(p0[0], p0[1], p1[0], p1[1], y10, y11, dinv, b1, w_cat)


def _combine2_body(a0_ref, a1_ref, y_ref, dinv_ref, b_ref, mu_ref, lv_ref):
    dinv = dinv_ref[:, 0:1]
    o = dinv * (a0_ref[...] + a1_ref[...] - y_ref[...]) + b_ref[...]
    mu_ref[...] = o[:, :16]
    lv_ref[...] = o[:, 16:]


def _combine2(a0, a1, y2, dinv, b_cat):
    half = pl.BlockSpec((_BROW, 16), lambda i: (i, 0))
    return pl.pallas_call(
        _combine2_body,
        grid=(_NP // _BROW,),
        in_specs=[pl.BlockSpec((_BROW, 32), lambda i: (i, 0)),
                  pl.BlockSpec((_BROW, 32), lambda i: (i, 0)),
                  pl.BlockSpec((_BROW, 32), lambda i: (i, 0)),
                  pl.BlockSpec((_BROW, 16), lambda i: (i, 0)),
                  pl.BlockSpec((1, 32), lambda i: (0, 0))],
        out_specs=[half, half],
        out_shape=[jax.ShapeDtypeStruct((_NP, 16), jnp.float32),
                   jax.ShapeDtypeStruct((_NP, 16), jnp.float32)],
        compiler_params=pltpu.CompilerParams(
            dimension_semantics=("parallel",)),
    )(a0, a1, y2, dinv, b_cat)


_BM = 512
_BN = 2048


def _decoder_body(z_row_ref, z_col_ref, out_ref):
    a = z_row_ref[...] * 0.5
    acc = jax.lax.dot_general(a, z_col_ref[...],
                              (((1,), (1,)), ((), ())),
                              preferred_element_type=jnp.float32)
    out_ref[...] = jnp.tanh(acc) * 0.5 + 0.5


def _decoder(z):
    return pl.pallas_call(
        _decoder_body,
        grid=(pl.cdiv(_N, _BM), pl.cdiv(_N, _BN)),
        in_specs=[pl.BlockSpec((_BM, 16), lambda i, j: (i, 0)),
                  pl.BlockSpec((_BN, 16), lambda i, j: (j, 0))],
        out_specs=pl.BlockSpec((_BM, _BN), lambda i, j: (i, j)),
        out_shape=jax.ShapeDtypeStruct((_N, _N), jnp.float32),
        compiler_params=pltpu.CompilerParams(
            dimension_semantics=("parallel", "parallel")),
    )(z, z)


def kernel(x, edge_index, W1, b1, W_mu, b_mu, W_lv, b_lv):
    ei = edge_index.astype(jnp.int32)
    src = jnp.concatenate([ei[0], jnp.zeros((_EP - _E,), jnp.int32)])
    dst = jnp.concatenate([ei[1],
                           jnp.full((_EP - _E,), _N, jnp.int32)])
    src3 = src.reshape(_NW, _NCHUNK, _K)
    dst3 = dst.reshape(_NW, _NCHUNK, _K)

    ones = jnp.ones((_K, 16), jnp.float32)
    zeros = jnp.zeros((_NP, 16), jnp.float32)
    x_pad = jnp.pad(x, ((0, _NP - _N), (0, 0)))
    w_cat = jnp.concatenate([W_mu, W_lv], axis=1)
    b_cat = jnp.concatenate([b_mu, b_lv]).reshape(1, 32)

    deg = _deg_kernel(dst3, ones, zeros)
    y10, y11, dinv = _mmscale(x_pad, W1, deg[0], deg[1])
    p0, p1 = _prop128(y10, y11, src3, dst3)
    y2 = _combine1(p0, p1, y10, y11, dinv, b1.reshape(1, 128), w_cat)
    p2 = _prop32(y2, src3, dst3)
    mu_p, lv_p = _combine2(p2[0], p2[1], y2, dinv, b_cat)
    adj_pred = _decoder(mu_p)
    return (adj_pred, mu_p[:_N], lv_p[:_N])
